# baseline (speedup 1.0000x reference)
import functools
import jax
import jax.numpy as jnp
import numpy as np
from jax import lax
from jax.experimental import pallas as pl
from jax.experimental.pallas import tpu as pltpu
from jax.experimental.pallas import tpu_sc as plsc

K = 16
N = 10000
D = 128
E = 320000

NSUB = 32
CHUNK = E // NSUB
VECS = CHUNK // 16
ROWS = 80
LROWS = 160
BATCH = 64
COPYR = 64
NPAD = 10240
NHALF = NPAD // 2
AROWS = NHALF + BATCH
ADUMP = NHALF
SLAB = NHALF // 16
DEAD = 16

_CP = pltpu.CompilerParams(needs_layout_passes=False)
_MESH = plsc.VectorSubcoreMesh(core_axis_name="c", subcore_axis_name="s")

_rand = np.asarray(jax.random.uniform(jax.random.key(42), (E,),
                                      dtype=jnp.float32))
_order = np.argsort(_rand, kind="stable").astype(np.int32)
_ORDER = jnp.asarray(
    np.pad(_order.reshape(NSUB, CHUNK), ((0, 0), (0, ROWS * 128 - CHUNK)))
    .reshape(NSUB, ROWS, 128))


def _vz16i():
    return jnp.zeros((16,), jnp.int32)


def _vz16f():
    return jnp.zeros((16,), jnp.float32)


def _k1_body(src_hbm, dst_hbm, order_hbm,
             hist_hbm, hsrc_hbm, packed_hbm, srco_hbm,
             order_v, dsto_v, srco_v, srcc_v, cnt_v, cnts_v, pk_v, sem):
    w = lax.axis_index("c") * 16 + lax.axis_index("s")
    base = w * CHUNK

    pltpu.sync_copy(order_hbm.at[w], order_v)
    for j in range(ROWS):
        pltpu.async_copy(dst_hbm.at[order_v.at[j]], dsto_v.at[j], sem)
    for j in range(ROWS):
        pltpu.async_copy(src_hbm.at[order_v.at[j]], srco_v.at[j], sem)
    pltpu.sync_copy(src_hbm.at[pl.ds(base, CHUNK)], srcc_v)

    def zero(i, _):
        o = pl.multiple_of(i * 16, 16)
        cnt_v[pl.ds(o, 16)] = _vz16i()
        cnts_v[pl.ds(o, 16)] = _vz16i()
        return 0

    lax.fori_loop(0, NPAD // 16, zero, 0)

    pltpu.make_async_copy(order_hbm.at[w], dsto_v, sem).wait()
    pltpu.make_async_copy(order_hbm.at[w], srco_v, sem).wait()

    def count_dst(i, _):
        row = i >> 3
        col = (i & 7) * 16
        n = dsto_v[row, pl.ds(col, 16)]
        lcnt, last = plsc.scan_count(n)
        old = plsc.load_gather(cnt_v, [n])
        new = old + lcnt
        plsc.store_scatter(cnt_v, [n], new, mask=last)
        lr = jnp.minimum(new - 1, DEAD)
        pk_v[row, pl.ds(col, 16)] = (n << 5) | lr
        return 0

    lax.fori_loop(0, VECS, count_dst, 0)

    def count_src(i, _):
        o = pl.multiple_of(i * 16, 16)
        n = srcc_v[pl.ds(o, 16)]
        lcnt, last = plsc.scan_count(n)
        old = plsc.load_gather(cnts_v, [n])
        plsc.store_scatter(cnts_v, [n], old + lcnt, mask=last)
        return 0

    lax.fori_loop(0, VECS, count_src, 0)

    pltpu.sync_copy(cnt_v, hist_hbm.at[w])
    pltpu.sync_copy(cnts_v, hsrc_hbm.at[w])
    pltpu.sync_copy(pk_v, packed_hbm.at[w])
    pltpu.sync_copy(srco_v, srco_hbm.at[w])


_k1 = pl.kernel(
    _k1_body,
    out_type=(
        jax.ShapeDtypeStruct((NSUB, NPAD), jnp.int32),
        jax.ShapeDtypeStruct((NSUB, NPAD), jnp.int32),
        jax.ShapeDtypeStruct((NSUB, ROWS, 128), jnp.int32),
        jax.ShapeDtypeStruct((NSUB, ROWS, 128), jnp.int32),
    ),
    scratch_types=[
        pltpu.VMEM((ROWS, 128), jnp.int32),
        pltpu.VMEM((ROWS, 128), jnp.int32),
        pltpu.VMEM((ROWS, 128), jnp.int32),
        pltpu.VMEM((CHUNK,), jnp.int32),
        pltpu.VMEM((NPAD,), jnp.int32),
        pltpu.VMEM((NPAD,), jnp.int32),
        pltpu.VMEM((ROWS, 128), jnp.int32),
        pltpu.SemaphoreType.DMA,
    ],
    mesh=_MESH,
    compiler_params=_CP,
)


def _k2_body(x_ref, hs_ref, hd_ref, feat_ref, normb_ref):
    od = jnp.sum(hs_ref[...], axis=0).astype(jnp.float32)
    scale = lax.rsqrt(jnp.maximum(od, 1.0))
    feat_ref[...] = x_ref[...] * scale[:, None]
    it = jnp.sum(hd_ref[...], axis=0).astype(jnp.float32)
    nm = lax.rsqrt(jnp.clip(jnp.maximum(it, 1.0), 0.0, float(K)))
    normb_ref[...] = jnp.broadcast_to(nm[:, None], (1024, D))


def _k2(xpad, hsrc, hist):
    return pl.pallas_call(
        _k2_body,
        grid=(NPAD // 1024,),
        in_specs=[
            pl.BlockSpec((1024, D), lambda i: (i, 0)),
            pl.BlockSpec((NSUB, 1024), lambda i: (0, i)),
            pl.BlockSpec((NSUB, 1024), lambda i: (0, i)),
        ],
        out_specs=[
            pl.BlockSpec((1024, D), lambda i: (i, 0)),
            pl.BlockSpec((1024, D), lambda i: (i, 0)),
        ],
        out_shape=[
            jax.ShapeDtypeStruct((NPAD, D), jnp.float32),
            jax.ShapeDtypeStruct((NPAD, D), jnp.float32),
        ],
    )(xpad, hsrc, hist)


def _k3_body(hist_hbm, packed_hbm, srco_hbm, feat_hbm,
             agg_hbm,
             off_v, hrow_v, pk_v, srco_v,
             buf0, buf1, buf2, buf3,
             agg_sh,
             gs0, gs1, gs2, gs3):
    cid = lax.axis_index("c")
    sid = lax.axis_index("s")
    nbase = cid * NHALF
    bufs = (buf0, buf1, buf2, buf3)
    gsems = (gs0, gs1, gs2, gs3)

    def zrows(i, _):
        buf0[i >> 3, pl.ds((i & 7) * 16, 16)] = _vz16f()
        return 0

    lax.fori_loop(0, COPYR * 8, zrows, 0)

    for j in range(SLAB // COPYR):
        pltpu.sync_copy(buf0.at[pl.ds(0, COPYR)],
                        agg_sh.at[pl.ds(sid * SLAB + j * COPYR, COPYR)])

    def zoff(i, _):
        o = pl.multiple_of(i * 16, 16)
        off_v[pl.ds(o, 16)] = _vz16i()
        return 0

    lax.fori_loop(0, NHALF // 16, zoff, 0)

    def accum_hist(lo, hi):
        def prefix(r, _):
            pltpu.sync_copy(hist_hbm.at[r, pl.ds(nbase, NHALF)], hrow_v)

            def acc(i, _):
                o = pl.multiple_of(i * 16, 16)
                off_v[pl.ds(o, 16)] = (off_v[pl.ds(o, 16)]
                                       + hrow_v[pl.ds(o, 16)])
                return 0

            lax.fori_loop(0, NHALF // 16, acc, 0)
            return 0

        lax.fori_loop(lo, hi, prefix, 0)

    plsc.subcore_barrier()

    def do_chunk(w):
        pltpu.sync_copy(packed_hbm.at[w], pk_v)
        pltpu.sync_copy(srco_hbm.at[w], srco_v)

        def rank(i, nk):
            row = i >> 2
            col = (i & 3) * 16
            pk = pk_v[row, pl.ds(col, 16)]
            sv = srco_v[row, pl.ds(col, 16)]
            n = pk >> 5
            lr = pk & 31
            rel = n - nbase
            inhalf = (rel >= 0) & (rel < NHALF)
            reli = jnp.clip(rel, 0, NHALF - 1)
            g = plsc.load_gather(off_v, [reli]) + lr
            keep = (g < K) & inhalf
            ics = plsc.cumsum(jnp.where(keep, 1, 0), mask=keep)
            pos = jnp.where(keep, nk + ics - 1, 0)
            plsc.store_scatter(srco_v, [pos >> 6, pos & 63], sv, mask=keep)
            plsc.store_scatter(pk_v, [pos >> 6, pos & 63], reli, mask=keep)
            return nk + jnp.max(ics)

        nk = lax.fori_loop(0, VECS, rank, jnp.int32(0))

        nbatch = (nk + BATCH - 1) // BATCH
        nb4 = (nbatch + 3) >> 2
        v0 = nk >> 4
        lane = lax.broadcasted_iota(jnp.int32, (16,), 0)
        pad = lane >= (nk & 15)
        row0 = v0 >> 2
        col0 = (v0 & 3) * 16
        cur_s = srco_v[row0, pl.ds(col0, 16)]
        cur_p = pk_v[row0, pl.ds(col0, 16)]
        srco_v[row0, pl.ds(col0, 16)] = jnp.where(pad, 0, cur_s)
        pk_v[row0, pl.ds(col0, 16)] = jnp.where(pad, ADUMP, cur_p)

        def fill(v, _):
            row = v >> 2
            col = (v & 3) * 16
            srco_v[row, pl.ds(col, 16)] = _vz16i()
            pk_v[row, pl.ds(col, 16)] = jnp.full((16,), ADUMP, jnp.int32)
            return 0

        lax.fori_loop(v0 + 1, 4 * nb4 * (BATCH // 16), fill, 0)

        def quad(b4, _):
            b = 4 * b4
            gets = [
                pltpu.async_copy(feat_hbm.at[srco_v.at[b + u]],
                                 bufs[u], gsems[u])
                for u in range(4)
            ]
            for g in gets:
                g.wait()
            for u in range(4):
                pltpu.sync_copy(bufs[u], agg_sh.at[pk_v.at[b + u]], add=True)
            return 0

        lax.fori_loop(0, nb4, quad, 0)

    accum_hist(0, sid)
    do_chunk(sid)
    accum_hist(sid, sid + 16)
    do_chunk(sid + 16)

    plsc.subcore_barrier()

    for j in range(SLAB // COPYR):
        sl = pl.ds(sid * SLAB + j * COPYR, COPYR)
        pltpu.sync_copy(agg_sh.at[sl], agg_hbm.at[cid].at[sl])


_k3 = pl.kernel(
    _k3_body,
    out_type=jax.ShapeDtypeStruct((2, NHALF, D), jnp.float32),
    scratch_types=[
        pltpu.VMEM((NHALF,), jnp.int32),
        pltpu.VMEM((NHALF,), jnp.int32),
        pltpu.VMEM((LROWS, BATCH), jnp.int32),
        pltpu.VMEM((LROWS, BATCH), jnp.int32),
        pltpu.VMEM((BATCH, D), jnp.float32),
        pltpu.VMEM((BATCH, D), jnp.float32),
        pltpu.VMEM((BATCH, D), jnp.float32),
        pltpu.VMEM((BATCH, D), jnp.float32),
        pltpu.VMEM_SHARED((AROWS, D), jnp.float32),
        pltpu.SemaphoreType.DMA,
        pltpu.SemaphoreType.DMA,
        pltpu.SemaphoreType.DMA,
        pltpu.SemaphoreType.DMA,
    ],
    mesh=_MESH,
    compiler_params=_CP,
)


def _k4_body(a_ref, nb_ref, w_ref, b_ref, out_ref):
    a = a_ref[...] * nb_ref[...]
    out_ref[...] = jnp.dot(a, w_ref[...],
                           preferred_element_type=jnp.float32) + b_ref[...]


def _k4(agg, normb, weight, bias2d):
    return pl.pallas_call(
        _k4_body,
        grid=(NPAD // 1024,),
        in_specs=[
            pl.BlockSpec((1024, D), lambda i: (i, 0)),
            pl.BlockSpec((1024, D), lambda i: (i, 0)),
            pl.BlockSpec((D, D), lambda i: (0, 0)),
            pl.BlockSpec((1, D), lambda i: (0, 0)),
        ],
        out_specs=pl.BlockSpec((1024, D), lambda i: (i, 0)),
        out_shape=jax.ShapeDtypeStruct((NPAD, D), jnp.float32),
    )(agg, normb, weight, bias2d)


def kernel(x, edge_index, weight, bias):
    src = edge_index[0]
    dst = edge_index[1]
    hist, hsrc, packed, srco = _k1(src, dst, _ORDER)
    xpad = jnp.concatenate([x, jnp.zeros((NPAD - N, D), x.dtype)])
    feat, normb = _k2(xpad, hsrc, hist)
    agg = _k3(hist,
              packed.reshape(NSUB, LROWS, BATCH),
              srco.reshape(NSUB, LROWS, BATCH),
              feat).reshape(NPAD, D)
    out = _k4(agg, normb, weight, bias.reshape(1, D))
    return out[:N]


# speedup vs baseline: 1.4662x; 1.4662x over previous
import functools
import jax
import jax.numpy as jnp
import numpy as np
from jax import lax
from jax.experimental import pallas as pl
from jax.experimental.pallas import tpu as pltpu
from jax.experimental.pallas import tpu_sc as plsc

K = 16
N = 10000
D = 128
E = 320000

NSUB = 32
CHUNK = E // NSUB
VECS = CHUNK // 16
ROWS = 80
LROWS = 160
BATCH = 64
COPYR = 64
NPAD = 10240
NHALF = NPAD // 2
AROWS = NHALF + BATCH
ADUMP = NHALF
SLAB = NHALF // 16
DEAD = 16

_CP = pltpu.CompilerParams(needs_layout_passes=False)
_MESH = plsc.VectorSubcoreMesh(core_axis_name="c", subcore_axis_name="s")

_rand = np.asarray(jax.random.uniform(jax.random.key(42), (E,),
                                      dtype=jnp.float32))
_order = np.argsort(_rand, kind="stable").astype(np.int32)
_ORDER = jnp.asarray(
    np.pad(_order.reshape(NSUB, CHUNK), ((0, 0), (0, ROWS * 128 - CHUNK)))
    .reshape(NSUB, ROWS, 128))


def _vz16i():
    return jnp.zeros((16,), jnp.int32)


def _vz16f():
    return jnp.zeros((16,), jnp.float32)


def _k1_body(src_hbm, dst_hbm, order_hbm,
             hist_hbm, hsrc_hbm, packed_hbm, srco_hbm,
             order_v, dsto_v, srco_v, srcc_v, cnt_v, cnts_v, pk_v, sem):
    w = lax.axis_index("c") * 16 + lax.axis_index("s")
    base = w * CHUNK

    pltpu.sync_copy(order_hbm.at[w], order_v)
    for j in range(ROWS):
        pltpu.async_copy(dst_hbm.at[order_v.at[j]], dsto_v.at[j], sem)
    for j in range(ROWS):
        pltpu.async_copy(src_hbm.at[order_v.at[j]], srco_v.at[j], sem)
    pltpu.sync_copy(src_hbm.at[pl.ds(base, CHUNK)], srcc_v)

    def zero(i, _):
        o = pl.multiple_of(i * 16, 16)
        cnt_v[pl.ds(o, 16)] = _vz16i()
        cnts_v[pl.ds(o, 16)] = _vz16i()
        return 0

    lax.fori_loop(0, NPAD // 16, zero, 0)

    pltpu.make_async_copy(order_hbm.at[w], dsto_v, sem).wait()
    pltpu.make_async_copy(order_hbm.at[w], srco_v, sem).wait()

    def count_dst(i, _):
        row = i >> 3
        col = (i & 7) * 16
        n = dsto_v[row, pl.ds(col, 16)]
        lcnt, last = plsc.scan_count(n)
        old = plsc.load_gather(cnt_v, [n])
        new = old + lcnt
        plsc.store_scatter(cnt_v, [n], new, mask=last)
        lr = jnp.minimum(new - 1, DEAD)
        pk_v[row, pl.ds(col, 16)] = (n << 5) | lr
        return 0

    lax.fori_loop(0, VECS, count_dst, 0)

    def count_src(i, _):
        o = pl.multiple_of(i * 16, 16)
        n = srcc_v[pl.ds(o, 16)]
        lcnt, last = plsc.scan_count(n)
        old = plsc.load_gather(cnts_v, [n])
        plsc.store_scatter(cnts_v, [n], old + lcnt, mask=last)
        return 0

    lax.fori_loop(0, VECS, count_src, 0)

    pltpu.sync_copy(cnt_v, hist_hbm.at[w])
    pltpu.sync_copy(cnts_v, hsrc_hbm.at[w])
    pltpu.sync_copy(pk_v, packed_hbm.at[w])
    pltpu.sync_copy(srco_v, srco_hbm.at[w])


_k1 = pl.kernel(
    _k1_body,
    out_type=(
        jax.ShapeDtypeStruct((NSUB, NPAD), jnp.int32),
        jax.ShapeDtypeStruct((NSUB, NPAD), jnp.int32),
        jax.ShapeDtypeStruct((NSUB, ROWS, 128), jnp.int32),
        jax.ShapeDtypeStruct((NSUB, ROWS, 128), jnp.int32),
    ),
    scratch_types=[
        pltpu.VMEM((ROWS, 128), jnp.int32),
        pltpu.VMEM((ROWS, 128), jnp.int32),
        pltpu.VMEM((ROWS, 128), jnp.int32),
        pltpu.VMEM((CHUNK,), jnp.int32),
        pltpu.VMEM((NPAD,), jnp.int32),
        pltpu.VMEM((NPAD,), jnp.int32),
        pltpu.VMEM((ROWS, 128), jnp.int32),
        pltpu.SemaphoreType.DMA,
    ],
    mesh=_MESH,
    compiler_params=_CP,
)


def _k2_body(x_ref, hs_ref, hd_ref, feat_ref, normb_ref):
    od = jnp.sum(hs_ref[...], axis=0).astype(jnp.float32)
    scale = lax.rsqrt(jnp.maximum(od, 1.0))
    feat_ref[...] = x_ref[...] * scale[:, None]
    it = jnp.sum(hd_ref[...], axis=0).astype(jnp.float32)
    nm = lax.rsqrt(jnp.clip(jnp.maximum(it, 1.0), 0.0, float(K)))
    normb_ref[...] = jnp.broadcast_to(nm[:, None], (1024, D))


def _k2(xpad, hsrc, hist):
    return pl.pallas_call(
        _k2_body,
        grid=(NPAD // 1024,),
        in_specs=[
            pl.BlockSpec((1024, D), lambda i: (i, 0)),
            pl.BlockSpec((NSUB, 1024), lambda i: (0, i)),
            pl.BlockSpec((NSUB, 1024), lambda i: (0, i)),
        ],
        out_specs=[
            pl.BlockSpec((1024, D), lambda i: (i, 0)),
            pl.BlockSpec((1024, D), lambda i: (i, 0)),
        ],
        out_shape=[
            jax.ShapeDtypeStruct((NPAD, D), jnp.float32),
            jax.ShapeDtypeStruct((NPAD, D), jnp.float32),
        ],
    )(xpad, hsrc, hist)


def _k3_body(hist_hbm, packed_hbm, srco_hbm, feat_hbm,
             agg_hbm,
             off_v, hrow_v, pk_v, srco_v,
             buf0,
             agg_sh,
             gs0):
    cid = lax.axis_index("c")
    sid = lax.axis_index("s")
    nbase = cid * NHALF

    def zrows(i, _):
        buf0[i >> 3, pl.ds((i & 7) * 16, 16)] = _vz16f()
        return 0

    lax.fori_loop(0, COPYR * 8, zrows, 0)

    for j in range(SLAB // COPYR):
        pltpu.sync_copy(buf0.at[pl.ds(0, COPYR)],
                        agg_sh.at[pl.ds(sid * SLAB + j * COPYR, COPYR)])

    def zoff(i, _):
        o = pl.multiple_of(i * 16, 16)
        off_v[pl.ds(o, 16)] = _vz16i()
        return 0

    lax.fori_loop(0, NHALF // 16, zoff, 0)

    def accum_hist(lo, hi):
        def prefix(r, _):
            pltpu.sync_copy(hist_hbm.at[r, pl.ds(nbase, NHALF)], hrow_v)

            def acc(i, _):
                o = pl.multiple_of(i * 16, 16)
                off_v[pl.ds(o, 16)] = (off_v[pl.ds(o, 16)]
                                       + hrow_v[pl.ds(o, 16)])
                return 0

            lax.fori_loop(0, NHALF // 16, acc, 0)
            return 0

        lax.fori_loop(lo, hi, prefix, 0)

    plsc.subcore_barrier()

    def do_chunk(w):
        pltpu.sync_copy(packed_hbm.at[w], pk_v)
        pltpu.sync_copy(srco_hbm.at[w], srco_v)

        def rank(i, nk):
            row = i >> 2
            col = (i & 3) * 16
            pk = pk_v[row, pl.ds(col, 16)]
            sv = srco_v[row, pl.ds(col, 16)]
            n = pk >> 5
            lr = pk & 31
            rel = n - nbase
            inhalf = (rel >= 0) & (rel < NHALF)
            reli = jnp.clip(rel, 0, NHALF - 1)
            g = plsc.load_gather(off_v, [reli]) + lr
            keep = (g < K) & inhalf
            ics = plsc.cumsum(jnp.where(keep, 1, 0), mask=keep)
            pos = jnp.where(keep, nk + ics - 1, 0)
            plsc.store_scatter(srco_v, [pos >> 6, pos & 63], sv, mask=keep)
            plsc.store_scatter(pk_v, [pos >> 6, pos & 63], reli, mask=keep)
            return nk + jnp.max(ics)

        nk = lax.fori_loop(0, VECS, rank, jnp.int32(0))

        nbatch = (nk + BATCH - 1) // BATCH
        v0 = nk >> 4
        lane = lax.broadcasted_iota(jnp.int32, (16,), 0)
        pad = lane >= (nk & 15)
        row0 = v0 >> 2
        col0 = (v0 & 3) * 16
        cur_s = srco_v[row0, pl.ds(col0, 16)]
        cur_p = pk_v[row0, pl.ds(col0, 16)]
        srco_v[row0, pl.ds(col0, 16)] = jnp.where(pad, 0, cur_s)
        pk_v[row0, pl.ds(col0, 16)] = jnp.where(pad, ADUMP, cur_p)

        def fill(v, _):
            row = v >> 2
            col = (v & 3) * 16
            srco_v[row, pl.ds(col, 16)] = _vz16i()
            pk_v[row, pl.ds(col, 16)] = jnp.full((16,), ADUMP, jnp.int32)
            return 0

        lax.fori_loop(v0 + 1, nbatch * (BATCH // 16), fill, 0)

        def batch(b, _):
            pltpu.async_copy(feat_hbm.at[srco_v.at[b]], buf0, gs0).wait()
            pltpu.sync_copy(buf0, agg_sh.at[pk_v.at[b]], add=True)
            return 0

        lax.fori_loop(0, nbatch, batch, 0)

    accum_hist(0, sid)
    do_chunk(sid)
    accum_hist(sid, sid + 16)
    do_chunk(sid + 16)

    plsc.subcore_barrier()

    for j in range(SLAB // COPYR):
        sl = pl.ds(sid * SLAB + j * COPYR, COPYR)
        pltpu.sync_copy(agg_sh.at[sl], agg_hbm.at[cid].at[sl])


_k3 = pl.kernel(
    _k3_body,
    out_type=jax.ShapeDtypeStruct((2, NHALF, D), jnp.float32),
    scratch_types=[
        pltpu.VMEM((NHALF,), jnp.int32),
        pltpu.VMEM((NHALF,), jnp.int32),
        pltpu.VMEM((LROWS, BATCH), jnp.int32),
        pltpu.VMEM((LROWS, BATCH), jnp.int32),
        pltpu.VMEM((BATCH, D), jnp.float32),
        pltpu.VMEM_SHARED((AROWS, D), jnp.float32),
        pltpu.SemaphoreType.DMA,
    ],
    mesh=_MESH,
    compiler_params=_CP,
)


def _k4_body(a_ref, nb_ref, w_ref, b_ref, out_ref):
    a = a_ref[...] * nb_ref[...]
    out_ref[...] = jnp.dot(a, w_ref[...],
                           preferred_element_type=jnp.float32) + b_ref[...]


def _k4(agg, normb, weight, bias2d):
    return pl.pallas_call(
        _k4_body,
        grid=(NPAD // 1024,),
        in_specs=[
            pl.BlockSpec((1024, D), lambda i: (i, 0)),
            pl.BlockSpec((1024, D), lambda i: (i, 0)),
            pl.BlockSpec((D, D), lambda i: (0, 0)),
            pl.BlockSpec((1, D), lambda i: (0, 0)),
        ],
        out_specs=pl.BlockSpec((1024, D), lambda i: (i, 0)),
        out_shape=jax.ShapeDtypeStruct((NPAD, D), jnp.float32),
    )(agg, normb, weight, bias2d)


def kernel(x, edge_index, weight, bias):
    src = edge_index[0]
    dst = edge_index[1]
    hist, hsrc, packed, srco = _k1(src, dst, _ORDER)
    xpad = jnp.concatenate([x, jnp.zeros((NPAD - N, D), x.dtype)])
    feat, normb = _k2(xpad, hsrc, hist)
    agg = _k3(hist,
              packed.reshape(NSUB, LROWS, BATCH),
              srco.reshape(NSUB, LROWS, BATCH),
              feat).reshape(NPAD, D)
    out = _k4(agg, normb, weight, bias.reshape(1, D))
    return out[:N]
